# baseline (speedup 1.0000x reference)
import dataclasses
import functools

import jax
import jax.numpy as jnp
from jax import lax
from jax.experimental import pallas as pl
from jax.experimental.pallas import tpu as pltpu
from jax.experimental.pallas import tpu_sc as plsc

N = 100000
R = 4
LOG2C = 13
C = 1 << LOG2C
NBLK = 14
BLK_PER_SC = 7
NSUB = 16
G = 256
QUANT = NSUB * G
ROWS_PER_SUB = C // NSUB


def _bucketize(edge_index, etypes):
    src = edge_index[0].astype(jnp.int32)
    dst = edge_index[1].astype(jnp.int32)
    et = etypes.astype(jnp.int32)
    E = src.shape[0]
    blk = dst >> LOG2C
    order = jnp.argsort(blk)
    counts = jnp.bincount(blk, length=NBLK).astype(jnp.int32)
    padded = ((counts + QUANT - 1) // QUANT) * QUANT
    pstart = jnp.concatenate(
        [jnp.zeros(1, jnp.int32), jnp.cumsum(padded)[:-1].astype(jnp.int32)])
    start = jnp.concatenate(
        [jnp.zeros(1, jnp.int32), jnp.cumsum(counts)[:-1].astype(jnp.int32)])
    blk_s = blk[order]
    rank = jnp.arange(E, dtype=jnp.int32) - start[blk_s]
    pos = pstart[blk_s] + rank
    g = (et * N + src)[order]
    l = (dst & (C - 1))[order]
    cap = E + NBLK * QUANT
    gidx = jnp.zeros(cap, jnp.int32).at[pos].set(g)
    ldst = jnp.full(cap, C, jnp.int32).at[pos].set(l)
    ar = jnp.arange(32, dtype=jnp.int32)
    kk = ar & 15
    cc = ar >> 4
    bid = jnp.clip(cc * BLK_PER_SC + kk, 0, NBLK - 1)
    offs = jnp.where(kk < BLK_PER_SC, pstart[bid], 0).astype(jnp.int32)
    cnts = jnp.where(kk < BLK_PER_SC, padded[bid], 0).astype(jnp.int32)
    return gidx, ldst, offs, cnts


def _masked_scalar(vec, k):
    io = lax.iota(jnp.int32, 16)
    return jnp.sum(jnp.where(io == k, vec, 0))


@functools.partial(jax.jit, static_argnames=("dout",))
def _sc_aggregate(xr_flat, gidx, ldst, offs, cnts, zeros_hbm, dout):
    mesh = plsc.VectorSubcoreMesh(core_axis_name="c", subcore_axis_name="s")
    cp = pltpu.CompilerParams()
    fields = pltpu.CompilerParams.__dataclass_fields__
    if "needs_layout_passes" in fields:
        cp = dataclasses.replace(cp, needs_layout_passes=False)
    if "use_tc_tiling_on_sc" in fields:
        cp = dataclasses.replace(cp, use_tc_tiling_on_sc=False)

    @functools.partial(
        pl.kernel,
        out_type=jax.ShapeDtypeStruct((NBLK * C, dout), jnp.float32),
        mesh=mesh,
        compiler_params=cp,
        scratch_types=[
            pltpu.VMEM((G,), jnp.int32),
            pltpu.VMEM((G,), jnp.int32),
            pltpu.VMEM((G, dout), jnp.float32),
            pltpu.VMEM((16,), jnp.int32),
            pltpu.VMEM((16,), jnp.int32),
            pltpu.VMEM_SHARED((C + 8, dout), jnp.float32),
            pltpu.SemaphoreType.DMA,
        ],
    )
    def k(xr_hbm, gidx_hbm, ldst_hbm, offs_hbm, cnts_hbm, z_hbm, agg_hbm,
          idxv, ldstv, rows, offv, cntv, acc, sem):
        core = lax.axis_index("c")
        sub = lax.axis_index("s")
        pltpu.sync_copy(offs_hbm.at[pl.ds(core * 16, 16)], offv)
        pltpu.sync_copy(cnts_hbm.at[pl.ds(core * 16, 16)], cntv)
        zslc = pl.ds(sub * ROWS_PER_SUB, ROWS_PER_SUB)
        pltpu.sync_copy(z_hbm.at[zslc], acc.at[zslc])
        plsc.subcore_barrier()
        for kb in range(BLK_PER_SC):
            startv = _masked_scalar(offv[...], kb)
            cntk = _masked_scalar(cntv[...], kb)
            per = lax.shift_right_logical(cntk, 4)
            trips = lax.shift_right_logical(per, 8)
            mybase = startv + sub * per

            def body(j, carry):
                off = pl.multiple_of(mybase + j * G, G)
                pltpu.sync_copy(gidx_hbm.at[pl.ds(off, G)], idxv)
                pltpu.sync_copy(ldst_hbm.at[pl.ds(off, G)], ldstv)
                pltpu.async_copy(xr_hbm.at[idxv], rows, sem).wait()
                pltpu.sync_copy(rows, acc.at[ldstv], add=True)
                return carry

            lax.fori_loop(0, trips, body, 0)
            plsc.subcore_barrier()
            blkid = core * BLK_PER_SC + kb
            out_slc = pl.ds(blkid * C + sub * ROWS_PER_SUB, ROWS_PER_SUB)
            pltpu.sync_copy(acc.at[zslc], agg_hbm.at[out_slc])
            pltpu.sync_copy(z_hbm.at[zslc], acc.at[zslc])
            plsc.subcore_barrier()

    return k(xr_flat, gidx, ldst, offs, cnts, zeros_hbm)


def _matmul_all(h, Wall):
    n, din = h.shape
    nr, _, dout = Wall.shape
    bn = 1000
    nblocks = n // bn

    def body(h_ref, w_ref, o_ref):
        o_ref[...] = lax.dot_general(
            h_ref[...], w_ref[0],
            dimension_numbers=(((1,), (0,)), ((), ())),
            preferred_element_type=jnp.float32,
            precision=lax.Precision.HIGHEST,
        )

    return pl.pallas_call(
        body,
        grid=(nr, nblocks),
        in_specs=[
            pl.BlockSpec((bn, din), lambda r, i: (i, 0)),
            pl.BlockSpec((1, din, dout), lambda r, i: (r, 0, 0)),
        ],
        out_specs=pl.BlockSpec((bn, dout), lambda r, i: (r * nblocks + i, 0)),
        out_shape=jax.ShapeDtypeStruct((nr * n, dout), jnp.float32),
    )(h, Wall)


def _finalize(agg, selfm, b):
    n, dout = agg.shape
    bn = 1000
    nblocks = n // bn

    def body(a_ref, s_ref, b_ref, o_ref):
        o_ref[...] = jnp.maximum(a_ref[...] + s_ref[...] + b_ref[...], 0.0)

    return pl.pallas_call(
        body,
        grid=(nblocks,),
        in_specs=[
            pl.BlockSpec((bn, dout), lambda i: (i, 0)),
            pl.BlockSpec((bn, dout), lambda i: (i, 0)),
            pl.BlockSpec((1, dout), lambda i: (0, 0)),
        ],
        out_specs=pl.BlockSpec((bn, dout), lambda i: (i, 0)),
        out_shape=jax.ShapeDtypeStruct((n, dout), jnp.float32),
    )(agg, selfm, b.reshape(1, dout))


def kernel(x, edge_index, etypes, W1, Ws1, b1, W2, Ws2, b2, W3, Ws3, b3):
    gidx, ldst, offs, cnts = _bucketize(edge_index, etypes)
    h = x
    for (W, Ws, b) in ((W1, Ws1, b1), (W2, Ws2, b2), (W3, Ws3, b3)):
        dout = W.shape[2]
        Wall = jnp.concatenate([W, Ws[None]], axis=0)
        xr = _matmul_all(h, Wall)
        zeros_hbm = jnp.zeros((C, dout), jnp.float32)
        agg = _sc_aggregate(xr, gidx, ldst, offs, cnts, zeros_hbm, dout)
        h = _finalize(agg[:N], xr[R * N:(R + 1) * N], b)
    return h


# speedup vs baseline: 2.1867x; 2.1867x over previous
import dataclasses
import functools

import jax
import jax.numpy as jnp
from jax import lax
from jax.experimental import pallas as pl
from jax.experimental.pallas import tpu as pltpu
from jax.experimental.pallas import tpu_sc as plsc

N = 100000
R = 4
LOG2C = 13
C = 1 << LOG2C
NBLK = 14
BLK_PER_SC = 7
NSUB = 16
NW = 2 * NSUB
LOG2G = 8
G = 1 << LOG2G
ROWS_PER_SUB = C // NSUB
E_TOT = 1600000
CAP = E_TOT + NBLK * NW * G
HTILE = 10000


def _sc_compiler_params():
    cp = pltpu.CompilerParams()
    fields = pltpu.CompilerParams.__dataclass_fields__
    if "needs_layout_passes" in fields:
        cp = dataclasses.replace(cp, needs_layout_passes=False)
    if "use_tc_tiling_on_sc" in fields:
        cp = dataclasses.replace(cp, use_tc_tiling_on_sc=False)
    return cp


def _masked_scalar(vec, k):
    io = lax.iota(jnp.int32, 16)
    return jnp.sum(jnp.where(io == k, vec, 0))


def _lane_select(k, add_vec):
    io = lax.iota(jnp.int32, 16)
    return jnp.where(io == k, add_vec, 0)


@jax.jit
def _sc_hist(dst):
    E = dst.shape[0]
    eps = E // NW
    ntiles = eps // HTILE
    mesh = plsc.VectorSubcoreMesh(core_axis_name="c", subcore_axis_name="s")

    @functools.partial(
        pl.kernel,
        out_type=jax.ShapeDtypeStruct((NW * 16,), jnp.int32),
        mesh=mesh,
        compiler_params=_sc_compiler_params(),
        scratch_types=[
            pltpu.VMEM((HTILE,), jnp.int32),
            pltpu.VMEM((16,), jnp.int32),
        ],
    )
    def k(dst_hbm, hist_hbm, dv, hv):
        core = lax.axis_index("c")
        sub = lax.axis_index("s")
        wid = core * NSUB + sub
        base = wid * eps
        io = lax.iota(jnp.int32, 16)

        def tile(i, acc):
            off = pl.multiple_of(base + i * HTILE, 8)
            pltpu.sync_copy(dst_hbm.at[pl.ds(off, HTILE)], dv)

            def vreg(j, acc2):
                d = dv[pl.ds(pl.multiple_of(j * 16, 16), 16)]
                b = lax.shift_right_logical(d, LOG2C)
                for kk in range(NBLK):
                    pc = plsc.all_reduce_population_count(b == kk)
                    acc2 = acc2 + jnp.where(io == kk, pc, 0)
                return acc2

            return lax.fori_loop(0, HTILE // 16, vreg, acc)

        acc = lax.fori_loop(0, ntiles, tile, jnp.zeros((16,), jnp.int32))
        hv[...] = acc
        pltpu.sync_copy(
            hv, hist_hbm.at[pl.ds(pl.multiple_of(wid * 16, 16), 16)])

    return k(dst)


@jax.jit
def _sc_bucketize(src, dst, et, hist):
    E = src.shape[0]
    eps = E // NW
    ntiles = eps // HTILE
    cap = E + NBLK * NW * G
    mesh = plsc.VectorSubcoreMesh(core_axis_name="c", subcore_axis_name="s")

    @functools.partial(
        pl.kernel,
        out_type=(
            jax.ShapeDtypeStruct((cap,), jnp.int32),
            jax.ShapeDtypeStruct((cap,), jnp.int32),
            jax.ShapeDtypeStruct((NW * 16,), jnp.int32),
            jax.ShapeDtypeStruct((NW * 16,), jnp.int32),
        ),
        mesh=mesh,
        compiler_params=_sc_compiler_params(),
        scratch_types=[
            pltpu.VMEM((HTILE,), jnp.int32),
            pltpu.VMEM((HTILE,), jnp.int32),
            pltpu.VMEM((HTILE,), jnp.int32),
            pltpu.VMEM((NW * 16,), jnp.int32),
            pltpu.VMEM((NBLK, 2 * G), jnp.int32),
            pltpu.VMEM((NBLK, 2 * G), jnp.int32),
            pltpu.VMEM((16,), jnp.int32),
            pltpu.VMEM((16,), jnp.int32),
        ],
    )
    def k(src_hbm, dst_hbm, et_hbm, hist_hbm,
          gidx_hbm, ldst_hbm, soff_hbm, strips_hbm,
          sv, dv, ev, hvm, stg, stl, rowa, rowb):
        core = lax.axis_index("c")
        sub = lax.axis_index("s")
        wid = core * NSUB + sub
        io = lax.iota(jnp.int32, 16)
        pltpu.sync_copy(hist_hbm, hvm)

        def ph_row(w):
            h = hvm[pl.ds(w * 16, 16)]
            return lax.shift_left(
                lax.shift_right_logical(h + (G - 1), LOG2G), LOG2G)

        tot = jnp.zeros((16,), jnp.int32)
        for w in range(NW):
            tot = tot + ph_row(w)
        pstart = jnp.cumsum(tot) - tot
        myoff = pstart
        for w in range(NW):
            myoff = myoff + ph_row(w) * (w < wid).astype(jnp.int32)
        hmy = hvm[pl.ds(pl.multiple_of(wid * 16, 16), 16)]
        myph = lax.shift_left(
            lax.shift_right_logical(hmy + (G - 1), LOG2G), LOG2G)
        rowa[...] = myoff
        rowb[...] = lax.shift_right_logical(myph, LOG2G)
        pltpu.sync_copy(rowa, soff_hbm.at[pl.ds(pl.multiple_of(wid * 16, 16), 16)])
        pltpu.sync_copy(rowb, strips_hbm.at[pl.ds(pl.multiple_of(wid * 16, 16), 16)])

        base = wid * eps

        def tile(i, carry):
            wptrv, flshv = carry
            off = pl.multiple_of(base + i * HTILE, 8)
            pltpu.sync_copy(src_hbm.at[pl.ds(off, HTILE)], sv)
            pltpu.sync_copy(dst_hbm.at[pl.ds(off, HTILE)], dv)
            pltpu.sync_copy(et_hbm.at[pl.ds(off, HTILE)], ev)

            def vreg(j, carry2):
                wptrv2, flshv2 = carry2
                slc = pl.ds(pl.multiple_of(j * 16, 16), 16)
                d = dv[slc]
                s = sv[slc]
                e = ev[slc]
                b = lax.shift_right_logical(d, LOG2C)
                gv = e * N + s
                lv = d & (C - 1)
                for kk in range(NBLK):
                    m = b == kk
                    pc = plsc.all_reduce_population_count(m)
                    ranks = plsc.cumsum(m.astype(jnp.int32)) - 1
                    wk = _masked_scalar(wptrv2, kk)
                    pos = wk + ranks
                    plsc.store_scatter(stg.at[kk], [pos], gv, mask=m)
                    plsc.store_scatter(stl.at[kk], [pos], lv, mask=m)
                    pcs = _masked_scalar(pc, 0)
                    new_wk = wk + pcs
                    wptrv2 = wptrv2 + _lane_select(kk, pc)

                    def flush(ops):
                        wv, fv = ops
                        fk = _masked_scalar(fv, kk)
                        sk = _masked_scalar(myoff, kk)
                        dsto = pl.multiple_of(sk + fk, G)
                        pltpu.sync_copy(stg.at[kk, pl.ds(0, G)],
                                        gidx_hbm.at[pl.ds(dsto, G)])
                        pltpu.sync_copy(stl.at[kk, pl.ds(0, G)],
                                        ldst_hbm.at[pl.ds(dsto, G)])
                        tg = stg[kk, pl.ds(G, 16)]
                        tl = stl[kk, pl.ds(G, 16)]
                        stg[kk, pl.ds(0, 16)] = tg
                        stl[kk, pl.ds(0, 16)] = tl
                        return (wv - _lane_select(kk, jnp.full((16,), G, jnp.int32)),
                                fv + _lane_select(kk, jnp.full((16,), G, jnp.int32)))

                    wptrv2, flshv2 = lax.cond(
                        new_wk >= G, flush, lambda ops: ops, (wptrv2, flshv2))
                return (wptrv2, flshv2)

            return lax.fori_loop(0, HTILE // 16, vreg, (wptrv, flshv))

        zero16 = jnp.zeros((16,), jnp.int32)
        wptrv, flshv = lax.fori_loop(0, ntiles, tile, (zero16, zero16))

        for kk in range(NBLK):
            wk = _masked_scalar(wptrv, kk)
            for j in range(G // 16):
                slc = pl.ds(j * 16, 16)
                mpad = (io + j * 16) >= wk
                stg[kk, slc] = jnp.where(mpad, 0, stg[kk, slc])
                stl[kk, slc] = jnp.where(mpad, C, stl[kk, slc])

            @pl.when(wk > 0)
            def _():
                fk = _masked_scalar(flshv, kk)
                sk = _masked_scalar(myoff, kk)
                dsto = pl.multiple_of(sk + fk, G)
                pltpu.sync_copy(stg.at[kk, pl.ds(0, G)],
                                gidx_hbm.at[pl.ds(dsto, G)])
                pltpu.sync_copy(stl.at[kk, pl.ds(0, G)],
                                ldst_hbm.at[pl.ds(dsto, G)])

    return k(src, dst, et, hist)


@functools.partial(jax.jit, static_argnames=("dout",))
def _sc_aggregate(xr_flat, gidx, ldst, soff, strips, zeros_hbm, dout):
    mesh = plsc.VectorSubcoreMesh(core_axis_name="c", subcore_axis_name="s")

    @functools.partial(
        pl.kernel,
        out_type=jax.ShapeDtypeStruct((NBLK * C, dout), jnp.float32),
        mesh=mesh,
        compiler_params=_sc_compiler_params(),
        scratch_types=[
            pltpu.VMEM((G,), jnp.int32),
            pltpu.VMEM((G,), jnp.int32),
            pltpu.VMEM((G, dout), jnp.float32),
            pltpu.VMEM((16,), jnp.int32),
            pltpu.VMEM((16,), jnp.int32),
            pltpu.VMEM_SHARED((C + 8, dout), jnp.float32),
            pltpu.SemaphoreType.DMA,
        ],
    )
    def k(xr_hbm, gidx_hbm, ldst_hbm, soff_hbm, strips_hbm, z_hbm, agg_hbm,
          idxv, ldstv, rows, offv, trv, acc, sem):
        core = lax.axis_index("c")
        sub = lax.axis_index("s")
        zslc = pl.ds(sub * ROWS_PER_SUB, ROWS_PER_SUB)
        pltpu.sync_copy(z_hbm.at[zslc], acc.at[zslc])
        plsc.subcore_barrier()
        for kb in range(BLK_PER_SC):
            blkid = core * BLK_PER_SC + kb
            for seg in range(2):
                w = 2 * sub + seg
                wslc = pl.ds(pl.multiple_of(w * 16, 16), 16)
                pltpu.sync_copy(soff_hbm.at[wslc], offv)
                pltpu.sync_copy(strips_hbm.at[wslc], trv)
                segoff = _masked_scalar(offv[...], blkid)
                trips = _masked_scalar(trv[...], blkid)

                def body(j, carry):
                    off = pl.multiple_of(segoff + j * G, G)
                    pltpu.sync_copy(gidx_hbm.at[pl.ds(off, G)], idxv)
                    pltpu.sync_copy(ldst_hbm.at[pl.ds(off, G)], ldstv)
                    pltpu.async_copy(xr_hbm.at[idxv], rows, sem).wait()
                    pltpu.sync_copy(rows, acc.at[ldstv], add=True)
                    return carry

                lax.fori_loop(0, trips, body, 0)
            plsc.subcore_barrier()
            out_slc = pl.ds(blkid * C + sub * ROWS_PER_SUB, ROWS_PER_SUB)
            pltpu.sync_copy(acc.at[zslc], agg_hbm.at[out_slc])
            pltpu.sync_copy(z_hbm.at[zslc], acc.at[zslc])
            plsc.subcore_barrier()

    return k(xr_flat, gidx, ldst, soff, strips, zeros_hbm)


def _matmul_all(h, Wall):
    n, din = h.shape
    nr, _, dout = Wall.shape
    bn = 1000
    nblocks = n // bn

    def body(h_ref, w_ref, o_ref):
        o_ref[...] = lax.dot_general(
            h_ref[...], w_ref[0],
            dimension_numbers=(((1,), (0,)), ((), ())),
            preferred_element_type=jnp.float32,
            precision=lax.Precision.HIGHEST,
        )

    return pl.pallas_call(
        body,
        grid=(nr, nblocks),
        in_specs=[
            pl.BlockSpec((bn, din), lambda r, i: (i, 0)),
            pl.BlockSpec((1, din, dout), lambda r, i: (r, 0, 0)),
        ],
        out_specs=pl.BlockSpec((bn, dout), lambda r, i: (r * nblocks + i, 0)),
        out_shape=jax.ShapeDtypeStruct((nr * n, dout), jnp.float32),
    )(h, Wall)


def _finalize(agg, selfm, b):
    n, dout = agg.shape
    bn = 1000
    nblocks = n // bn

    def body(a_ref, s_ref, b_ref, o_ref):
        o_ref[...] = jnp.maximum(a_ref[...] + s_ref[...] + b_ref[...], 0.0)

    return pl.pallas_call(
        body,
        grid=(nblocks,),
        in_specs=[
            pl.BlockSpec((bn, dout), lambda i: (i, 0)),
            pl.BlockSpec((bn, dout), lambda i: (i, 0)),
            pl.BlockSpec((1, dout), lambda i: (0, 0)),
        ],
        out_specs=pl.BlockSpec((bn, dout), lambda i: (i, 0)),
        out_shape=jax.ShapeDtypeStruct((n, dout), jnp.float32),
    )(agg, selfm, b.reshape(1, dout))


def kernel(x, edge_index, etypes, W1, Ws1, b1, W2, Ws2, b2, W3, Ws3, b3):
    src = edge_index[0].astype(jnp.int32)
    dst = edge_index[1].astype(jnp.int32)
    et = etypes.astype(jnp.int32)
    hist = _sc_hist(dst)
    gidx, ldst, soff, strips = _sc_bucketize(src, dst, et, hist)
    h = x
    for (W, Ws, b) in ((W1, Ws1, b1), (W2, Ws2, b2), (W3, Ws3, b3)):
        dout = W.shape[2]
        Wall = jnp.concatenate([W, Ws[None]], axis=0)
        xr = _matmul_all(h, Wall)
        zeros_hbm = jnp.zeros((C, dout), jnp.float32)
        agg = _sc_aggregate(xr, gidx, ldst, soff, strips, zeros_hbm, dout)
        h = _finalize(agg[:N], xr[R * N:(R + 1) * N], b)
    return h


# speedup vs baseline: 2.2885x; 1.0465x over previous
import dataclasses
import functools

import jax
import jax.numpy as jnp
from jax import lax
from jax.experimental import pallas as pl
from jax.experimental.pallas import tpu as pltpu
from jax.experimental.pallas import tpu_sc as plsc

N = 100000
R = 4
LOG2C = 13
C = 1 << LOG2C
NBLK = 14
BLK_PER_SC = 7
NSUB = 16
NW = 2 * NSUB
LOG2G = 8
G = 1 << LOG2G
ROWS_PER_SUB = C // NSUB
E_TOT = 1600000
CAP = E_TOT + NBLK * NW * G
HTILE = 10000


def _sc_compiler_params():
    cp = pltpu.CompilerParams()
    fields = pltpu.CompilerParams.__dataclass_fields__
    if "needs_layout_passes" in fields:
        cp = dataclasses.replace(cp, needs_layout_passes=False)
    if "use_tc_tiling_on_sc" in fields:
        cp = dataclasses.replace(cp, use_tc_tiling_on_sc=False)
    return cp


def _masked_scalar(vec, k):
    io = lax.iota(jnp.int32, 16)
    return jnp.sum(jnp.where(io == k, vec, 0))


def _lane_select(k, add_vec):
    io = lax.iota(jnp.int32, 16)
    return jnp.where(io == k, add_vec, 0)


@jax.jit
def _sc_hist(dst):
    E = dst.shape[0]
    eps = E // NW
    ntiles = eps // HTILE
    mesh = plsc.VectorSubcoreMesh(core_axis_name="c", subcore_axis_name="s")

    @functools.partial(
        pl.kernel,
        out_type=jax.ShapeDtypeStruct((NW * 16,), jnp.int32),
        mesh=mesh,
        compiler_params=_sc_compiler_params(),
        scratch_types=[
            pltpu.VMEM((HTILE,), jnp.int32),
            pltpu.VMEM((16,), jnp.int32),
        ],
    )
    def k(dst_hbm, hist_hbm, dv, hv):
        core = lax.axis_index("c")
        sub = lax.axis_index("s")
        wid = core * NSUB + sub
        base = wid * eps
        io = lax.iota(jnp.int32, 16)

        def tile(i, acc):
            off = pl.multiple_of(base + i * HTILE, 8)
            pltpu.sync_copy(dst_hbm.at[pl.ds(off, HTILE)], dv)

            def vreg(j, acc2):
                d = dv[pl.ds(pl.multiple_of(j * 16, 16), 16)]
                b = lax.shift_right_logical(d, LOG2C)
                for kk in range(NBLK):
                    pc = plsc.all_reduce_population_count(b == kk)
                    acc2 = acc2 + jnp.where(io == kk, pc, 0)
                return acc2

            return lax.fori_loop(0, HTILE // 16, vreg, acc)

        acc = lax.fori_loop(0, ntiles, tile, jnp.zeros((16,), jnp.int32))
        hv[...] = acc
        pltpu.sync_copy(
            hv, hist_hbm.at[pl.ds(pl.multiple_of(wid * 16, 16), 16)])

    return k(dst)


@jax.jit
def _sc_bucketize(src, dst, et, hist):
    E = src.shape[0]
    eps = E // NW
    ntiles = eps // HTILE
    cap = E + NBLK * NW * G
    mesh = plsc.VectorSubcoreMesh(core_axis_name="c", subcore_axis_name="s")

    @functools.partial(
        pl.kernel,
        out_type=(
            jax.ShapeDtypeStruct((cap,), jnp.int32),
            jax.ShapeDtypeStruct((cap,), jnp.int32),
            jax.ShapeDtypeStruct((32,), jnp.int32),
        ),
        mesh=mesh,
        compiler_params=_sc_compiler_params(),
        scratch_types=[
            pltpu.VMEM((HTILE,), jnp.int32),
            pltpu.VMEM((HTILE,), jnp.int32),
            pltpu.VMEM((HTILE,), jnp.int32),
            pltpu.VMEM((NW * 16,), jnp.int32),
            pltpu.VMEM((NBLK, 2 * G), jnp.int32),
            pltpu.VMEM((NBLK, 2 * G), jnp.int32),
            pltpu.VMEM((16,), jnp.int32),
            pltpu.VMEM((16,), jnp.int32),
        ],
    )
    def k(src_hbm, dst_hbm, et_hbm, hist_hbm,
          gidx_hbm, ldst_hbm, pq_hbm,
          sv, dv, ev, hvm, stg, stl, rowa, rowb):
        core = lax.axis_index("c")
        sub = lax.axis_index("s")
        wid = core * NSUB + sub
        io = lax.iota(jnp.int32, 16)
        pltpu.sync_copy(hist_hbm, hvm)

        def ph_row(w):
            h = hvm[pl.ds(w * 16, 16)]
            return lax.shift_left(
                lax.shift_right_logical(h + (G - 1), LOG2G), LOG2G)

        tot = jnp.zeros((16,), jnp.int32)
        for w in range(NW):
            tot = tot + ph_row(w)
        pstart = jnp.cumsum(tot) - tot
        myoff = pstart
        for w in range(NW):
            myoff = myoff + ph_row(w) * (w < wid).astype(jnp.int32)
        @pl.when(wid == 0)
        def _():
            rowa[...] = pstart
            rowb[...] = lax.shift_right_logical(tot, LOG2G)
            pltpu.sync_copy(rowa, pq_hbm.at[pl.ds(0, 16)])
            pltpu.sync_copy(rowb, pq_hbm.at[pl.ds(16, 16)])

        base = wid * eps

        def tile(i, carry):
            wptrv, flshv = carry
            off = pl.multiple_of(base + i * HTILE, 8)
            pltpu.sync_copy(src_hbm.at[pl.ds(off, HTILE)], sv)
            pltpu.sync_copy(dst_hbm.at[pl.ds(off, HTILE)], dv)
            pltpu.sync_copy(et_hbm.at[pl.ds(off, HTILE)], ev)

            def vreg(j, carry2):
                wptrv2, flshv2 = carry2
                slc = pl.ds(pl.multiple_of(j * 16, 16), 16)
                d = dv[slc]
                s = sv[slc]
                e = ev[slc]
                b = lax.shift_right_logical(d, LOG2C)
                gv = e * N + s
                lv = d & (C - 1)
                for kk in range(NBLK):
                    m = b == kk
                    pc = plsc.all_reduce_population_count(m)
                    ranks = plsc.cumsum(m.astype(jnp.int32)) - 1
                    wk = _masked_scalar(wptrv2, kk)
                    pos = wk + ranks
                    plsc.store_scatter(stg.at[kk], [pos], gv, mask=m)
                    plsc.store_scatter(stl.at[kk], [pos], lv, mask=m)
                    pcs = _masked_scalar(pc, 0)
                    new_wk = wk + pcs
                    wptrv2 = wptrv2 + _lane_select(kk, pc)

                    def flush(ops):
                        wv, fv = ops
                        fk = _masked_scalar(fv, kk)
                        sk = _masked_scalar(myoff, kk)
                        dsto = pl.multiple_of(sk + fk, G)
                        pltpu.sync_copy(stg.at[kk, pl.ds(0, G)],
                                        gidx_hbm.at[pl.ds(dsto, G)])
                        pltpu.sync_copy(stl.at[kk, pl.ds(0, G)],
                                        ldst_hbm.at[pl.ds(dsto, G)])
                        tg = stg[kk, pl.ds(G, 16)]
                        tl = stl[kk, pl.ds(G, 16)]
                        stg[kk, pl.ds(0, 16)] = tg
                        stl[kk, pl.ds(0, 16)] = tl
                        return (wv - _lane_select(kk, jnp.full((16,), G, jnp.int32)),
                                fv + _lane_select(kk, jnp.full((16,), G, jnp.int32)))

                    wptrv2, flshv2 = lax.cond(
                        new_wk >= G, flush, lambda ops: ops, (wptrv2, flshv2))
                return (wptrv2, flshv2)

            return lax.fori_loop(0, HTILE // 16, vreg, (wptrv, flshv))

        zero16 = jnp.zeros((16,), jnp.int32)
        wptrv, flshv = lax.fori_loop(0, ntiles, tile, (zero16, zero16))

        for kk in range(NBLK):
            wk = _masked_scalar(wptrv, kk)
            for j in range(G // 16):
                slc = pl.ds(j * 16, 16)
                mpad = (io + j * 16) >= wk
                stg[kk, slc] = jnp.where(mpad, 0, stg[kk, slc])
                stl[kk, slc] = jnp.where(mpad, C, stl[kk, slc])

            @pl.when(wk > 0)
            def _():
                fk = _masked_scalar(flshv, kk)
                sk = _masked_scalar(myoff, kk)
                dsto = pl.multiple_of(sk + fk, G)
                pltpu.sync_copy(stg.at[kk, pl.ds(0, G)],
                                gidx_hbm.at[pl.ds(dsto, G)])
                pltpu.sync_copy(stl.at[kk, pl.ds(0, G)],
                                ldst_hbm.at[pl.ds(dsto, G)])

    return k(src, dst, et, hist)


HW = 64


@functools.partial(jax.jit, static_argnames=("dout",))
def _sc_aggregate(xr_half, gidx, ldst, pq, zeros_hbm, dout):
    nh = dout // HW
    mesh = plsc.VectorSubcoreMesh(core_axis_name="c", subcore_axis_name="s")

    @functools.partial(
        pl.kernel,
        out_type=tuple(
            jax.ShapeDtypeStruct((NBLK * C, HW), jnp.float32)
            for _ in range(nh)),
        mesh=mesh,
        compiler_params=_sc_compiler_params(),
        scratch_types=[
            pltpu.VMEM((G,), jnp.int32),
            pltpu.VMEM((G,), jnp.int32),
            pltpu.VMEM((G,), jnp.int32),
            pltpu.VMEM((G,), jnp.int32),
            pltpu.VMEM((G,), jnp.int32),
            pltpu.VMEM((G,), jnp.int32),
            pltpu.VMEM((G, HW), jnp.float32),
            pltpu.VMEM((G, HW), jnp.float32),
            pltpu.VMEM((16,), jnp.int32),
            pltpu.VMEM((16,), jnp.int32),
            pltpu.VMEM_SHARED((C + 8, HW), jnp.float32),
            pltpu.SemaphoreType.DMA,
            pltpu.SemaphoreType.DMA,
            pltpu.SemaphoreType.DMA,
            pltpu.SemaphoreType.DMA,
            pltpu.SemaphoreType.DMA,
            pltpu.SemaphoreType.DMA,
        ],
    )
    def k(xr_hbm, gidx_hbm, ldst_hbm, pq_hbm, z_hbm, *rest):
        aggs = rest[:nh]
        (ix0, ix1, gx0, gx1, ld0, ld1, rw0, rw1, pv, qv, acc,
         sAg0, sAg1, sAl0, sAl1, sB0, sB1) = rest[nh:]
        core = lax.axis_index("c")
        sub = lax.axis_index("s")
        ix = (ix0, ix1)
        gx = (gx0, gx1) if nh > 1 else (ix0, ix1)
        ld = (ld0, ld1)
        rw = (rw0, rw1)
        sAg = (sAg0, sAg1)
        sAl = (sAl0, sAl1)
        sB = (sB0, sB1)
        pltpu.sync_copy(pq_hbm.at[pl.ds(0, 16)], pv)
        pltpu.sync_copy(pq_hbm.at[pl.ds(16, 16)], qv)
        zslc = pl.ds(sub * ROWS_PER_SUB, ROWS_PER_SUB)
        pltpu.sync_copy(z_hbm.at[zslc], acc.at[zslc])
        plsc.subcore_barrier()

        def block(kb, carry):
            blkid = core * BLK_PER_SC + kb
            pk = _masked_scalar(pv[...], blkid)
            qk = _masked_scalar(qv[...], blkid)
            T = lax.shift_right_logical(qk + 15 - sub, 4)

            for h in range(nh):
                def chunk_off(t):
                    return pl.multiple_of(pk + (sub + 16 * t) * G, G)

                def idx_start(t, b):
                    o = chunk_off(t)
                    pltpu.make_async_copy(
                        gidx_hbm.at[pl.ds(o, G)], ix[b], sAg[b]).start()
                    pltpu.make_async_copy(
                        ldst_hbm.at[pl.ds(o, G)], ld[b], sAl[b]).start()

                def idx_wait(b):
                    pltpu.make_async_copy(
                        gidx_hbm.at[pl.ds(0, G)], ix[b], sAg[b]).wait()
                    pltpu.make_async_copy(
                        ldst_hbm.at[pl.ds(0, G)], ld[b], sAl[b]).wait()
                    if nh > 1:
                        for jv in range(G // 16):
                            s = pl.ds(jv * 16, 16)
                            gx[b][s] = ix[b][s] * nh + h

                def gather_start(b):
                    pltpu.make_async_copy(
                        xr_hbm.at[gx[b]], rw[b], sB[b]).start()

                def gather_wait(b):
                    pltpu.make_async_copy(
                        xr_hbm.at[gx[b]], rw[b], sB[b]).wait()

                def scat(b):
                    pltpu.sync_copy(rw[b], acc.at[ld[b]], add=True)

                @pl.when(T > 0)
                def _():
                    idx_start(0, 0)

                @pl.when(T > 1)
                def _():
                    idx_start(1, 1)

                @pl.when(T > 0)
                def _():
                    idx_wait(0)
                    gather_start(0)

                def body(jj, c2):
                    t1 = 2 * jj + 1
                    t2 = 2 * jj + 2
                    t3 = 2 * jj + 3
                    gather_wait(0)

                    @pl.when(t1 < T)
                    def _():
                        idx_wait(1)
                        gather_start(1)

                    scat(0)

                    @pl.when(t2 < T)
                    def _():
                        idx_start(t2, 0)

                    @pl.when(t1 < T)
                    def _():
                        gather_wait(1)

                        @pl.when(t2 < T)
                        def _():
                            idx_wait(0)
                            gather_start(0)

                        scat(1)

                        @pl.when(t3 < T)
                        def _():
                            idx_start(t3, 1)

                    return c2

                lax.fori_loop(0, lax.shift_right_logical(T + 1, 1), body, 0)
                plsc.subcore_barrier()
                orow = pl.multiple_of(
                    blkid * C + sub * ROWS_PER_SUB, ROWS_PER_SUB)
                out_slc = pl.ds(orow, ROWS_PER_SUB)
                pltpu.sync_copy(acc.at[zslc], aggs[h].at[out_slc])
                pltpu.sync_copy(z_hbm.at[zslc], acc.at[zslc])
                plsc.subcore_barrier()
            return carry

        lax.fori_loop(0, BLK_PER_SC, block, 0)

    return k(xr_half, gidx, ldst, pq, zeros_hbm)


def _matmul_all(h, Wall):
    n, din = h.shape
    nr, _, dout = Wall.shape
    bn = 1000
    nblocks = n // bn

    def body(h_ref, w_ref, o_ref):
        o_ref[...] = lax.dot_general(
            h_ref[...], w_ref[0],
            dimension_numbers=(((1,), (0,)), ((), ())),
            preferred_element_type=jnp.float32,
            precision=lax.Precision.HIGHEST,
        )

    return pl.pallas_call(
        body,
        grid=(nr, nblocks),
        in_specs=[
            pl.BlockSpec((bn, din), lambda r, i: (i, 0)),
            pl.BlockSpec((1, din, dout), lambda r, i: (r, 0, 0)),
        ],
        out_specs=pl.BlockSpec((bn, dout), lambda r, i: (r * nblocks + i, 0)),
        out_shape=jax.ShapeDtypeStruct((nr * n, dout), jnp.float32),
    )(h, Wall)


def _finalize(aggs, selfm, b):
    n, dout = selfm.shape
    nh = len(aggs)
    hw = dout // nh
    bn = 1000
    nblocks = n // bn

    def body(*refs):
        a_refs = refs[:nh]
        s_ref, b_ref, o_ref = refs[nh:]
        a = jnp.concatenate([r[...] for r in a_refs], axis=1)
        o_ref[...] = jnp.maximum(a + s_ref[...] + b_ref[...], 0.0)

    return pl.pallas_call(
        body,
        grid=(nblocks,),
        in_specs=[pl.BlockSpec((bn, hw), lambda i: (i, 0))
                  for _ in range(nh)] + [
            pl.BlockSpec((bn, dout), lambda i: (i, 0)),
            pl.BlockSpec((1, dout), lambda i: (0, 0)),
        ],
        out_specs=pl.BlockSpec((bn, dout), lambda i: (i, 0)),
        out_shape=jax.ShapeDtypeStruct((n, dout), jnp.float32),
    )(*aggs, selfm, b.reshape(1, dout))


def kernel(x, edge_index, etypes, W1, Ws1, b1, W2, Ws2, b2, W3, Ws3, b3):
    src = edge_index[0].astype(jnp.int32)
    dst = edge_index[1].astype(jnp.int32)
    et = etypes.astype(jnp.int32)
    hist = _sc_hist(dst)
    gidx, ldst, pq = _sc_bucketize(src, dst, et, hist)
    h = x
    for (W, Ws, b) in ((W1, Ws1, b1), (W2, Ws2, b2), (W3, Ws3, b3)):
        dout = W.shape[2]
        Wall = jnp.concatenate([W, Ws[None]], axis=0)
        xr = _matmul_all(h, Wall)
        xr_half = xr.reshape((R + 1) * N * (dout // HW), HW)
        zeros_hbm = jnp.zeros((C, HW), jnp.float32)
        aggs = _sc_aggregate(xr_half, gidx, ldst, pq, zeros_hbm, dout)
        aggs = [a[:N] for a in aggs]
        h = _finalize(aggs, xr[R * N:(R + 1) * N], b)
    return h
